# baseline (speedup 1.0000x reference)
import functools

import jax
import jax.numpy as jnp
from jax import lax
from jax.experimental import pallas as pl
from jax.experimental.pallas import tpu as pltpu
from jax.experimental.pallas import tpu_sc as plsc

_NC = 2
_NS = 16
_LANES = 16
_NW = _NC * _NS
_CHUNK = 128


def _sc_scatter(idx2d, zeros, n, k):
    z = n // _NS
    mesh = plsc.VectorSubcoreMesh(core_axis_name="c", subcore_axis_name="s")

    @functools.partial(
        pl.kernel,
        out_type=jax.ShapeDtypeStruct((_NC * n,), jnp.int32),
        mesh=mesh,
        scratch_types=[
            pltpu.VMEM((k, _CHUNK), jnp.int32),
            pltpu.VMEM((_CHUNK,), jnp.int32),
            pltpu.VMEM((z,), jnp.int32),
            pltpu.VMEM_SHARED((n,), jnp.int32),
        ],
    )
    def run(idx_hbm, zeros_hbm, out_hbm, idx_v, ones_v, bounce, acc):
        c = lax.axis_index("c")
        s = lax.axis_index("s")
        wid = c * _NS + s
        for j in range(_CHUNK // _LANES):
            ones_v[pl.ds(j * _LANES, _LANES)] = jnp.ones((_LANES,), jnp.int32)
        pltpu.sync_copy(zeros_hbm.at[pl.ds(s * z, z)], bounce)
        pltpu.sync_copy(bounce, acc.at[pl.ds(s * z, z)])
        pltpu.sync_copy(idx_hbm.at[pl.ds(wid * k, k), :], idx_v)
        plsc.subcore_barrier()

        def body(j, carry):
            pltpu.sync_copy(ones_v, acc.at[idx_v.at[j]], add=True)
            return carry

        lax.fori_loop(0, k, body, 0)
        plsc.subcore_barrier()
        pltpu.sync_copy(acc.at[pl.ds(s * z, z)], bounce)
        pltpu.sync_copy(bounce, out_hbm.at[pl.ds(c * n + s * z, z)])

    return run(idx2d, zeros)


def _tc_upsample(fea, w_up, b_up, w_cls_t, b_cls, ta, tb, block_rows):
    n, d_in = fea.shape
    d_up = w_up.shape[1]
    grid = (n // block_rows,)

    def body(fea_ref, wup_ref, bup_ref, wcls_ref, bcls_ref, ta_ref, tb_ref,
             out_ref, ex_ref, tgt_ref):
        x = fea_ref[...].astype(jnp.bfloat16)
        up = jnp.dot(x, wup_ref[...], preferred_element_type=jnp.float32)
        up = up + bup_ref[...]
        ex = jnp.dot(up.astype(jnp.bfloat16), wcls_ref[...],
                     preferred_element_type=jnp.float32)[:, :1] + bcls_ref[...]
        t = (ta_ref[...] + tb_ref[...]) > 0
        keep = jnp.logical_or(ex > 0.0, t)
        out_ref[...] = jnp.where(keep, up, 0.0)
        ex_ref[...] = ex
        tgt_ref[...] = t.astype(jnp.int32)

    return pl.pallas_call(
        body,
        grid=grid,
        in_specs=[
            pl.BlockSpec((block_rows, d_in), lambda i: (i, 0)),
            pl.BlockSpec((d_in, d_up), lambda i: (0, 0)),
            pl.BlockSpec((1, d_up), lambda i: (0, 0)),
            pl.BlockSpec((d_up, 128), lambda i: (0, 0)),
            pl.BlockSpec((1, 1), lambda i: (0, 0)),
            pl.BlockSpec((block_rows, 1), lambda i: (i, 0)),
            pl.BlockSpec((block_rows, 1), lambda i: (i, 0)),
        ],
        out_specs=[
            pl.BlockSpec((block_rows, d_up), lambda i: (i, 0)),
            pl.BlockSpec((block_rows, 1), lambda i: (i, 0)),
            pl.BlockSpec((block_rows, 1), lambda i: (i, 0)),
        ],
        out_shape=[
            jax.ShapeDtypeStruct((n, d_up), jnp.float32),
            jax.ShapeDtypeStruct((n, 1), jnp.float32),
            jax.ShapeDtypeStruct((n, 1), jnp.int32),
        ],
    )(fea, w_up, b_up, w_cls_t, b_cls, ta, tb)


def kernel(fea, kernel_map_idx, W_up, b_up, W_cls, b_cls):
    n, d_in = fea.shape
    m = kernel_map_idx.shape[0]

    per_worker = _NW * _CHUNK
    k = -(-m // per_worker)
    k = -(-k // 8) * 8
    m_pad = k * per_worker
    idx = kernel_map_idx.astype(jnp.int32)
    idx_padded = jnp.concatenate(
        [idx, jnp.broadcast_to(idx[0], (m_pad - m,))])
    idx2d = idx_padded.reshape(m_pad // _CHUNK, _CHUNK)

    zeros = jnp.zeros((n,), jnp.int32)
    counts = _sc_scatter(idx2d, zeros, n, k)
    ta = counts[:n].reshape(n, 1)
    tb = counts[n:].reshape(n, 1)

    w_cls_pad = jnp.zeros((W_cls.shape[0], 128), W_cls.dtype).at[:, :1].set(W_cls)
    fea_out, exist, tgt = _tc_upsample(
        fea,
        W_up.astype(jnp.bfloat16),
        b_up.reshape(1, -1),
        w_cls_pad.astype(jnp.bfloat16),
        b_cls.reshape(1, 1),
        ta,
        tb,
        block_rows=2000,
    )
    target = tgt.reshape(n).astype(jnp.bool_)
    return (fea_out, exist, target)
